# speedup vs baseline: 27.2509x; 27.2509x over previous
import numpy as np
import jax
import jax.numpy as jnp
from jax.experimental import pallas as pl

_N, _J, _K = 16384, 200, 5
_C = _J * _K
_BLOCK = 1024

_R_NP = (np.arange(_C)[None, :] // _K == np.arange(_J)[:, None]).astype(np.float32)


def _body(x_ref, r_ref, t0_ref, t1_ref, o_ref):
    xb = x_ref[...].astype(jnp.bfloat16)
    xrep = jax.lax.dot_general(
        xb, r_ref[...],
        dimension_numbers=(((1,), (0,)), ((), ())),
        preferred_element_type=jnp.float32,
    )
    o_ref[...] = jnp.where(xrep > 0.5, t1_ref[...], t0_ref[...])


def kernel(x, table):
    r = jnp.asarray(_R_NP, dtype=jnp.bfloat16)
    t0 = jnp.tile(table[0], _J).reshape(1, _C)
    t1 = jnp.tile(table[1], _J).reshape(1, _C)
    out2d = pl.pallas_call(
        _body,
        grid=(_N // _BLOCK,),
        in_specs=[
            pl.BlockSpec((_BLOCK, _J), lambda i: (i, 0)),
            pl.BlockSpec((_J, _C), lambda i: (0, 0)),
            pl.BlockSpec((1, _C), lambda i: (0, 0)),
            pl.BlockSpec((1, _C), lambda i: (0, 0)),
        ],
        out_specs=pl.BlockSpec((_BLOCK, _C), lambda i: (i, 0)),
        out_shape=jax.ShapeDtypeStruct((_N, _C), jnp.float32),
    )(x, r, t0, t1)
    return out2d.reshape(_N, _J, _K)


# speedup vs baseline: 515.2543x; 18.9078x over previous
import jax
import jax.numpy as jnp
from jax.experimental import pallas as pl
from jax.experimental.pallas import tpu as pltpu

_N, _J, _K = 16384, 200, 5
_BLOCK_I = 2048


def _body(t_ref, x_ref, o_ref):
    xb = x_ref[...] != 0
    for k in range(_K):
        o_ref[k * _J:(k + 1) * _J, :] = jnp.where(xb, t_ref[1, k], t_ref[0, k])


def kernel(x, table):
    xt = x.T
    o2 = pl.pallas_call(
        _body,
        grid=(_N // _BLOCK_I,),
        in_specs=[
            pl.BlockSpec(memory_space=pltpu.SMEM),
            pl.BlockSpec((_J, _BLOCK_I), lambda i: (0, i)),
        ],
        out_specs=pl.BlockSpec((_J * _K, _BLOCK_I), lambda i: (0, i)),
        out_shape=jax.ShapeDtypeStruct((_J * _K, _N), jnp.float32),
    )(table, xt)
    return o2.reshape(_K, _J, _N).transpose(2, 1, 0)


# speedup vs baseline: 529.2028x; 1.0271x over previous
import jax
import jax.numpy as jnp
from jax.experimental import pallas as pl
from jax.experimental.pallas import tpu as pltpu

_N, _J, _K = 16384, 200, 5
_BLOCK_I = 4096


def _body(t_ref, x_ref, o_ref):
    xb = x_ref[...] != 0
    for k in range(_K):
        o_ref[k * _J:(k + 1) * _J, :] = jnp.where(xb, t_ref[1, k], t_ref[0, k])


def kernel(x, table):
    xt = x.T
    o2 = pl.pallas_call(
        _body,
        grid=(_N // _BLOCK_I,),
        in_specs=[
            pl.BlockSpec(memory_space=pltpu.SMEM),
            pl.BlockSpec((_J, _BLOCK_I), lambda i: (0, i)),
        ],
        out_specs=pl.BlockSpec((_J * _K, _BLOCK_I), lambda i: (0, i)),
        out_shape=jax.ShapeDtypeStruct((_J * _K, _N), jnp.float32),
    )(table, xt)
    return o2.reshape(_K, _J, _N).transpose(2, 1, 0)
